# baseline (speedup 1.0000x reference)
import functools

import jax
import jax.numpy as jnp
from jax import lax
from jax.experimental import pallas as pl
from jax.experimental.pallas import tpu as pltpu
from jax.experimental.pallas import tpu_sc as plsc

D = 64
N_LOW = 8192
N_HIGH = 2048
VOCAB = 100000
NC, NS = 2, 16
NW = NC * NS

_N_LH = N_LOW + N_HIGH
_LH_PER_W = _N_LH // NW


def _gather_lh_body(idx_hbm, w_hbm, out_hbm, idx_v, rows_v, pk_v, sem):
    wid = lax.axis_index("s") * NC + lax.axis_index("c")
    base = wid * _LH_PER_W
    pltpu.sync_copy(idx_hbm.at[pl.ds(base, _LH_PER_W)], idx_v)
    pltpu.async_copy(w_hbm.at[idx_v], rows_v, sem).wait()

    def row(i2, c):
        for h in range(2):
            for j in range(D // 16):
                pk_v[i2, pl.ds(h * D + j * 16, 16)] = (
                    rows_v[2 * i2 + h, pl.ds(j * 16, 16)])
        return c

    lax.fori_loop(0, _LH_PER_W // 2, row, 0)
    pltpu.sync_copy(pk_v, out_hbm.at[pl.ds(base // 2, _LH_PER_W // 2)])


def _gather_lh(idx, w):
    mesh = plsc.VectorSubcoreMesh(core_axis_name="c", subcore_axis_name="s",
                                  num_cores=NC, num_subcores=NS)
    return pl.kernel(
        _gather_lh_body,
        out_type=jax.ShapeDtypeStruct((_N_LH // 2, 2 * D), jnp.float32),
        mesh=mesh,
        compiler_params=pltpu.CompilerParams(use_tc_tiling_on_sc=False),
        scratch_types=[
            pltpu.VMEM((_LH_PER_W,), jnp.int32),
            pltpu.VMEM((_LH_PER_W, D), jnp.float32),
            pltpu.VMEM((_LH_PER_W // 2, 2 * D), jnp.float32),
            pltpu.SemaphoreType.DMA,
        ],
    )(idx, w)


_BL = 1024
_GRID = N_LOW // _BL
_DA = D + 8


def _noise_body(low_ref, high_ref, out_ref, acc_ref):
    step = pl.program_id(0)

    @pl.when(step == 0)
    def _():
        acc_ref[...] = jnp.zeros_like(acc_ref)

    low = low_ref[...]
    lemb = low[:, :D]
    high = high_ref[...]
    inv_l = 1.0 / jnp.sqrt(jnp.sum(lemb * lemb, axis=1, keepdims=True))
    inv_h = 1.0 / jnp.sqrt(jnp.sum(high * high, axis=1))
    dot = lax.dot_general(lemb, high, (((1,), (1,)), ((), ())),
                          preferred_element_type=jnp.float32)
    cos = dot * inv_l * inv_h[None, :]
    m = jnp.max(cos, axis=1, keepdims=True)
    col = lax.broadcasted_iota(jnp.int32, cos.shape, 1)
    am = jnp.min(jnp.where(cos == m, col, N_HIGH), axis=1, keepdims=True)
    w = jnp.exp(m)
    e = jnp.where(col == am, w, 0.0)
    acc_ref[...] += lax.dot_general(e, low, (((0,), (0,)), ((), ())),
                                    preferred_element_type=jnp.float32)

    @pl.when(step == _GRID - 1)
    def _():
        acc = acc_ref[...]
        den = acc[:, D:D + 1]
        out_ref[...] = jnp.where(den > 0.0, acc[:, :D] / den, 0.0)


def _compute_noise(low_aug, high_emb, interpret=False):
    return pl.pallas_call(
        _noise_body,
        grid=(_GRID,),
        in_specs=[
            pl.BlockSpec((_BL, _DA), lambda i: (i, 0)),
            pl.BlockSpec((N_HIGH, D), lambda i: (0, 0)),
        ],
        out_specs=pl.BlockSpec((N_HIGH, D), lambda i: (0, 0)),
        out_shape=jax.ShapeDtypeStruct((N_HIGH, D), jnp.float32),
        scratch_shapes=[pltpu.VMEM((N_HIGH, _DA), jnp.float32)],
        interpret=interpret,
    )(low_aug, high_emb)


_TOK = 1024 * 200
_T_PER_W = _TOK // NW
_C = 320
_NCHUNK = _T_PER_W // _C


def _combine_body(x_hbm, rev_hbm, noise_hbm, w_hbm, out_hbm,
                  xall, rall, wr0, wr1, nr0, nr1, ostage, noise_sh,
                  sx, sw0, sw1, sn0, sn1):
    sid = lax.axis_index("s")
    wid = sid * NC + lax.axis_index("c")
    tbase = wid * _T_PER_W

    @pl.when(sid == 0)
    def _():
        pltpu.sync_copy(noise_hbm, noise_sh)

    pltpu.sync_copy(x_hbm.at[pl.ds(tbase, _T_PER_W)], xall)
    pltpu.async_copy(rev_hbm.at[xall], rall, sx).wait()
    plsc.subcore_barrier()

    wrs, nrs = (wr0, wr1), (nr0, nr1)
    sws, sns = (sw0, sw1), (sn0, sn1)

    def _issue(ci, b):
        off = ci * _C
        pltpu.async_copy(w_hbm.at[xall.at[pl.ds(off, _C)]], wrs[b], sws[b])
        pltpu.async_copy(noise_sh.at[rall.at[pl.ds(off, _C)]], nrs[b], sns[b])

    for b in (0, 1):
        _issue(b, b)

    def step(i2, carry):
        for b in (0, 1):
            ci = i2 * 2 + b
            pltpu.make_async_copy(
                w_hbm.at[pl.ds(0, _C)], wrs[b], sws[b]).wait()
            pltpu.make_async_copy(
                noise_sh.at[pl.ds(0, _C)], nrs[b], sns[b]).wait()

            def row(i2, c, _b=b):
                for h in range(2):
                    for j in range(D // 16):
                        sl = pl.ds(j * 16, 16)
                        osl = pl.ds(h * D + j * 16, 16)
                        ostage[i2, osl] = (wrs[_b][2 * i2 + h, sl]
                                           + nrs[_b][2 * i2 + h, sl])
                return c

            lax.fori_loop(0, _C // 2, row, 0)
            pltpu.sync_copy(ostage,
                            out_hbm.at[pl.ds((tbase + ci * _C) // 2, _C // 2)])

            @pl.when(ci + 2 < _NCHUNK)
            def _(ci=ci, b=b):
                _issue(ci + 2, b)
        return carry

    lax.fori_loop(0, _NCHUNK // 2, step, 0)


def _combine(x_flat, rev, noise, w):
    mesh = plsc.VectorSubcoreMesh(core_axis_name="c", subcore_axis_name="s",
                                  num_cores=NC, num_subcores=NS)
    return pl.kernel(
        _combine_body,
        out_type=jax.ShapeDtypeStruct((_TOK // 2, 2 * D), jnp.float32),
        mesh=mesh,
        compiler_params=pltpu.CompilerParams(use_tc_tiling_on_sc=False),
        scratch_types=[
            pltpu.VMEM((_T_PER_W,), jnp.int32),
            pltpu.VMEM((_T_PER_W,), jnp.int32),
            pltpu.VMEM((_C, D), jnp.float32),
            pltpu.VMEM((_C, D), jnp.float32),
            pltpu.VMEM((_C, D), jnp.float32),
            pltpu.VMEM((_C, D), jnp.float32),
            pltpu.VMEM((_C // 2, 2 * D), jnp.float32),
            pltpu.VMEM_SHARED((N_HIGH + 1, D), jnp.float32),
            pltpu.SemaphoreType.DMA,
            pltpu.SemaphoreType.DMA,
            pltpu.SemaphoreType.DMA,
            pltpu.SemaphoreType.DMA,
            pltpu.SemaphoreType.DMA,
        ],
    )(x_flat, rev, noise, w)


def kernel(x, reverse_high_freqs, low_freqs, high_freqs, W):
    idx_lh = jnp.concatenate([low_freqs, high_freqs])
    rows = _gather_lh(idx_lh, W).reshape(_N_LH, D)
    low_emb = rows[:N_LOW]
    high_emb = rows[N_LOW:]
    low_aug = jnp.concatenate(
        [low_emb, jnp.ones((N_LOW, _DA - D), jnp.float32)], axis=1)
    noise = _compute_noise(low_aug, high_emb)
    noise_full = jnp.concatenate(
        [noise, jnp.zeros((1, D), jnp.float32)], axis=0)
    out = _combine(x.reshape(-1), reverse_high_freqs, noise_full, W)
    return out.reshape(x.shape[0], x.shape[1], D)


# speedup vs baseline: 1.0346x; 1.0346x over previous
import functools

import jax
import jax.numpy as jnp
from jax import lax
from jax.experimental import pallas as pl
from jax.experimental.pallas import tpu as pltpu
from jax.experimental.pallas import tpu_sc as plsc

D = 64
N_LOW = 8192
N_HIGH = 2048
VOCAB = 100000
NC, NS = 2, 16
NW = NC * NS

_N_LH = N_LOW + N_HIGH
_LH_PER_W = _N_LH // NW


def _gather_lh_body(idx_hbm, w_hbm, out_hbm, idx_v, rows_v, sem):
    wid = lax.axis_index("s") * NC + lax.axis_index("c")
    base = wid * _LH_PER_W
    pltpu.sync_copy(idx_hbm.at[pl.ds(base, _LH_PER_W)], idx_v)
    pltpu.async_copy(w_hbm.at[idx_v], rows_v, sem).wait()
    pltpu.sync_copy(rows_v, out_hbm.at[pl.ds(base, _LH_PER_W)])


def _gather_lh(idx, w):
    mesh = plsc.VectorSubcoreMesh(core_axis_name="c", subcore_axis_name="s",
                                  num_cores=NC, num_subcores=NS)
    return pl.kernel(
        _gather_lh_body,
        out_type=jax.ShapeDtypeStruct((_N_LH, D), jnp.float32),
        mesh=mesh,
        compiler_params=pltpu.CompilerParams(use_tc_tiling_on_sc=False),
        scratch_types=[
            pltpu.VMEM((_LH_PER_W,), jnp.int32),
            pltpu.VMEM((_LH_PER_W, D), jnp.float32),
            pltpu.SemaphoreType.DMA,
        ],
    )(idx, w)


_BL = 1024
_GRID = N_LOW // _BL
_DA = D + 8


def _noise_body(low_ref, high_ref, out_ref, acc_ref):
    step = pl.program_id(0)

    @pl.when(step == 0)
    def _():
        acc_ref[...] = jnp.zeros_like(acc_ref)

    low = low_ref[...]
    lemb = low[:, :D]
    high = high_ref[...]
    inv_l = 1.0 / jnp.sqrt(jnp.sum(lemb * lemb, axis=1, keepdims=True))
    inv_h = 1.0 / jnp.sqrt(jnp.sum(high * high, axis=1))
    dot = lax.dot_general(lemb, high, (((1,), (1,)), ((), ())),
                          preferred_element_type=jnp.float32)
    cos = dot * inv_l * inv_h[None, :]
    m = jnp.max(cos, axis=1, keepdims=True)
    col = lax.broadcasted_iota(jnp.int32, cos.shape, 1)
    am = jnp.min(jnp.where(cos == m, col, N_HIGH), axis=1, keepdims=True)
    w = jnp.exp(m)
    e = jnp.where(col == am, w, 0.0)
    acc_ref[...] += lax.dot_general(e, low, (((0,), (0,)), ((), ())),
                                    preferred_element_type=jnp.float32)

    @pl.when(step == _GRID - 1)
    def _():
        acc = acc_ref[...]
        den = acc[:, D:D + 1]
        out_ref[...] = jnp.where(den > 0.0, acc[:, :D] / den, 0.0)


def _compute_noise(low_aug, high_emb, interpret=False):
    return pl.pallas_call(
        _noise_body,
        grid=(_GRID,),
        in_specs=[
            pl.BlockSpec((_BL, _DA), lambda i: (i, 0)),
            pl.BlockSpec((N_HIGH, D), lambda i: (0, 0)),
        ],
        out_specs=pl.BlockSpec((N_HIGH, D), lambda i: (0, 0)),
        out_shape=jax.ShapeDtypeStruct((N_HIGH, D), jnp.float32),
        scratch_shapes=[pltpu.VMEM((N_HIGH, _DA), jnp.float32)],
        interpret=interpret,
    )(low_aug, high_emb)


_TOK = 1024 * 200
_T_PER_W = _TOK // NW
_C = 320
_NCHUNK = _T_PER_W // _C


def _combine_body(x_hbm, rev_hbm, noise_hbm, w_hbm, out_hbm,
                  xall, rall, wr0, wr1, nr0, nr1, ostage, noise_sh,
                  sx, sw0, sw1, sn0, sn1):
    sid = lax.axis_index("s")
    wid = sid * NC + lax.axis_index("c")
    tbase = wid * _T_PER_W

    @pl.when(sid == 0)
    def _():
        pltpu.sync_copy(noise_hbm, noise_sh)

    pltpu.sync_copy(x_hbm.at[pl.ds(tbase, _T_PER_W)], xall)
    pltpu.async_copy(rev_hbm.at[xall], rall, sx).wait()
    plsc.subcore_barrier()

    wrs, nrs = (wr0, wr1), (nr0, nr1)
    sws, sns = (sw0, sw1), (sn0, sn1)

    def _issue(ci, b):
        off = ci * _C
        pltpu.async_copy(w_hbm.at[xall.at[pl.ds(off, _C)]], wrs[b], sws[b])
        pltpu.async_copy(noise_sh.at[rall.at[pl.ds(off, _C)]], nrs[b], sns[b])

    for b in (0, 1):
        _issue(b, b)

    def step(i2, carry):
        for b in (0, 1):
            ci = i2 * 2 + b
            pltpu.make_async_copy(
                w_hbm.at[pl.ds(0, _C)], wrs[b], sws[b]).wait()
            pltpu.make_async_copy(
                noise_sh.at[pl.ds(0, _C)], nrs[b], sns[b]).wait()

            def row(i2, c, _b=b):
                for h in range(2):
                    for j in range(D // 16):
                        sl = pl.ds(j * 16, 16)
                        osl = pl.ds(h * D + j * 16, 16)
                        ostage[i2, osl] = (wrs[_b][2 * i2 + h, sl]
                                           + nrs[_b][2 * i2 + h, sl])
                return c

            lax.fori_loop(0, _C // 2, row, 0)
            pltpu.sync_copy(ostage,
                            out_hbm.at[pl.ds((tbase + ci * _C) // 2, _C // 2)])

            @pl.when(ci + 2 < _NCHUNK)
            def _(ci=ci, b=b):
                _issue(ci + 2, b)
        return carry

    lax.fori_loop(0, _NCHUNK // 2, step, 0)


def _combine(x_flat, rev, noise, w):
    mesh = plsc.VectorSubcoreMesh(core_axis_name="c", subcore_axis_name="s",
                                  num_cores=NC, num_subcores=NS)
    return pl.kernel(
        _combine_body,
        out_type=jax.ShapeDtypeStruct((_TOK // 2, 2 * D), jnp.float32),
        mesh=mesh,
        compiler_params=pltpu.CompilerParams(use_tc_tiling_on_sc=False),
        scratch_types=[
            pltpu.VMEM((_T_PER_W,), jnp.int32),
            pltpu.VMEM((_T_PER_W,), jnp.int32),
            pltpu.VMEM((_C, D), jnp.float32),
            pltpu.VMEM((_C, D), jnp.float32),
            pltpu.VMEM((_C, D), jnp.float32),
            pltpu.VMEM((_C, D), jnp.float32),
            pltpu.VMEM((_C // 2, 2 * D), jnp.float32),
            pltpu.VMEM_SHARED((N_HIGH + 1, D), jnp.float32),
            pltpu.SemaphoreType.DMA,
            pltpu.SemaphoreType.DMA,
            pltpu.SemaphoreType.DMA,
            pltpu.SemaphoreType.DMA,
            pltpu.SemaphoreType.DMA,
        ],
    )(x_flat, rev, noise, w)


def kernel(x, reverse_high_freqs, low_freqs, high_freqs, W):
    idx_lh = jnp.concatenate([low_freqs, high_freqs])
    rows = _gather_lh(idx_lh, W)
    low_emb = rows[:N_LOW]
    high_emb = rows[N_LOW:]
    low_aug = jnp.concatenate(
        [low_emb, jnp.ones((N_LOW, _DA - D), jnp.float32)], axis=1)
    noise = _compute_noise(low_aug, high_emb)
    noise_full = jnp.concatenate(
        [noise, jnp.zeros((1, D), jnp.float32)], axis=0)
    out = _combine(x.reshape(-1), reverse_high_freqs, noise_full, W)
    return out.reshape(x.shape[0], x.shape[1], D)
